# Initial kernel scaffold; baseline (speedup 1.0000x reference)
import jax
import jax.numpy as jnp
from jax.experimental import pallas as pl


def kernel(src_feats, cj, ci, edge_index):
    raise NotImplementedError("write your pallas kernel here")


# speedup vs baseline: 7.4418x; 7.4418x over previous
import functools

import jax
import jax.numpy as jnp
from jax import lax
from jax.experimental import pallas as pl
from jax.experimental.pallas import tpu as pltpu
from jax.experimental.pallas import tpu_sc as plsc

NC = 2
NS = 16
NW = NC * NS
LANES = 16


def _pick_chunk(epw: int) -> int:
    for c in (128, 120, 112, 104, 96, 88, 80, 72, 64, 56, 48, 40, 32, 24, 16, 8):
        if epw % c == 0:
            return c
    raise ValueError(f"no valid chunk for {epw}")


def _mul_rows_kernel(x_ref, c_ref, o_ref):
    o_ref[...] = x_ref[...] * c_ref[...]


def _combine_kernel(p_ref, c_ref, o_ref):
    o_ref[...] = (p_ref[0] + p_ref[1]) * c_ref[...]


def _sc_aggregate(weighted, src2d, dst2d, n, d, chunk, nchunk_w, zrows):
    nchunks = src2d.shape[0]
    rows_per_tile = n // NS
    nzcopy = rows_per_tile // zrows
    mesh = plsc.VectorSubcoreMesh(core_axis_name="c", subcore_axis_name="s")

    @functools.partial(
        pl.kernel,
        mesh=mesh,
        out_type=jax.ShapeDtypeStruct((NC, n, d), jnp.float32),
        compiler_params=pltpu.CompilerParams(use_tc_tiling_on_sc=False),
        scratch_types=[
            pltpu.VMEM((nchunk_w, chunk), jnp.int32),
            pltpu.VMEM((nchunk_w, chunk), jnp.int32),
            pltpu.VMEM((chunk, d), jnp.float32),
            pltpu.VMEM((zrows, d), jnp.float32),
            pltpu.VMEM_SHARED((n, d), jnp.float32),
            pltpu.SemaphoreType.DMA,
        ],
    )
    def k(w_hbm, src_hbm, dst_hbm, out_hbm, src_v, dst_v, rows_v, znd_v, acc, sem):
        c = lax.axis_index("c")
        s = lax.axis_index("s")
        wid = c * NS + s
        base_chunk = wid * nchunk_w
        row0 = s * rows_per_tile

        pltpu.sync_copy(src_hbm.at[pl.ds(base_chunk, nchunk_w)], src_v)
        pltpu.sync_copy(dst_hbm.at[pl.ds(base_chunk, nchunk_w)], dst_v)

        zvec = jnp.zeros((LANES,), jnp.float32)

        def zrow(r, _):
            for col in range(0, d, LANES):
                znd_v[r, pl.ds(col, LANES)] = zvec
            return 0

        lax.fori_loop(0, zrows, zrow, 0)
        for kk in range(nzcopy):
            pltpu.sync_copy(znd_v, acc.at[pl.ds(row0 + kk * zrows, zrows)])
        plsc.subcore_barrier()

        def body(j, _):
            pltpu.async_copy(w_hbm.at[src_v.at[j]], rows_v, sem).wait()
            pltpu.sync_copy(rows_v, acc.at[dst_v.at[j]], add=True)
            return 0

        lax.fori_loop(0, nchunk_w, body, 0)
        plsc.subcore_barrier()

        for kk in range(nzcopy):
            r = row0 + kk * zrows
            pltpu.sync_copy(acc.at[pl.ds(r, zrows)], znd_v)
            pltpu.sync_copy(znd_v, out_hbm.at[c, pl.ds(r, zrows)])

    return k(weighted, src2d, dst2d)


def kernel(src_feats, cj, ci, edge_index):
    n, d = src_feats.shape
    e = edge_index.shape[1]
    epw = e // NW
    assert epw * NW == e
    chunk = _pick_chunk(epw)
    nchunk_w = epw // chunk
    zrows = n // NS
    for z in (128, 125, 120, 100, 80, 64, 50, 40, 25, 20, 16, 10, 8, 5, 4, 2, 1):
        if (n // NS) % z == 0:
            zrows = z
            break

    br = 1000 if n % 1000 == 0 else 8

    weighted = pl.pallas_call(
        _mul_rows_kernel,
        out_shape=jax.ShapeDtypeStruct((n, d), jnp.float32),
        grid=(n // br,),
        in_specs=[
            pl.BlockSpec((br, d), lambda i: (i, 0)),
            pl.BlockSpec((br, 1), lambda i: (i, 0)),
        ],
        out_specs=pl.BlockSpec((br, d), lambda i: (i, 0)),
    )(src_feats, cj)

    src2d = edge_index[0].reshape(e // chunk, chunk)
    dst2d = edge_index[1].reshape(e // chunk, chunk)
    partial = _sc_aggregate(weighted, src2d, dst2d, n, d, chunk, nchunk_w, zrows)

    out = pl.pallas_call(
        _combine_kernel,
        out_shape=jax.ShapeDtypeStruct((n, d), jnp.float32),
        grid=(n // br,),
        in_specs=[
            pl.BlockSpec((NC, br, d), lambda i: (0, i, 0)),
            pl.BlockSpec((br, 1), lambda i: (i, 0)),
        ],
        out_specs=pl.BlockSpec((br, d), lambda i: (i, 0)),
    )(partial, ci)
    return out


# speedup vs baseline: 10.6227x; 1.4274x over previous
import functools

import jax
import jax.numpy as jnp
from jax import lax
from jax.experimental import pallas as pl
from jax.experimental.pallas import tpu as pltpu
from jax.experimental.pallas import tpu_sc as plsc

NC = 2
NS = 16
NW = NC * NS
LANES = 16


def _pick_chunk(epw: int) -> int:
    for c in (128, 120, 112, 104, 96, 88, 80, 72, 64, 56, 48, 40, 32, 24, 16, 8):
        if epw % c == 0:
            return c
    raise ValueError(f"no valid chunk for {epw}")


def _mul_rows_kernel(x_ref, c_ref, o_ref):
    o_ref[...] = x_ref[...] * c_ref[...]


def _combine_kernel(p_ref, c_ref, o_ref):
    o_ref[...] = (p_ref[0] + p_ref[1]) * c_ref[...]


def _sc_aggregate(weighted, src2d, dst2d, n, d, chunk, nchunk_w, zrows):
    nchunks = src2d.shape[0]
    rows_per_tile = n // NS
    nzcopy = rows_per_tile // zrows
    mesh = plsc.VectorSubcoreMesh(core_axis_name="c", subcore_axis_name="s")

    @functools.partial(
        pl.kernel,
        mesh=mesh,
        out_type=jax.ShapeDtypeStruct((NC, n, d), jnp.float32),
        compiler_params=pltpu.CompilerParams(use_tc_tiling_on_sc=False),
        scratch_types=[
            pltpu.VMEM((nchunk_w, chunk), jnp.int32),
            pltpu.VMEM((nchunk_w, chunk), jnp.int32),
            pltpu.VMEM((2, chunk, d), jnp.float32),
            pltpu.VMEM((zrows, d), jnp.float32),
            pltpu.VMEM_SHARED((n, d), jnp.float32),
            pltpu.SemaphoreType.DMA,
            pltpu.SemaphoreType.DMA,
        ],
    )
    def k(w_hbm, src_hbm, dst_hbm, out_hbm, src_v, dst_v, rows_v, znd_v, acc,
          sem_a, sem_b):
        c = lax.axis_index("c")
        s = lax.axis_index("s")
        wid = c * NS + s
        base_chunk = wid * nchunk_w
        row0 = s * rows_per_tile

        pltpu.sync_copy(src_hbm.at[pl.ds(base_chunk, nchunk_w)], src_v)
        pltpu.sync_copy(dst_hbm.at[pl.ds(base_chunk, nchunk_w)], dst_v)

        zvec = jnp.zeros((LANES,), jnp.float32)

        def zrow(r, _):
            for col in range(0, d, LANES):
                znd_v[r, pl.ds(col, LANES)] = zvec
            return 0

        lax.fori_loop(0, zrows, zrow, 0)
        for kk in range(nzcopy):
            pltpu.sync_copy(znd_v, acc.at[pl.ds(row0 + kk * zrows, zrows)])
        plsc.subcore_barrier()

        def gather(j, b, sem):
            pltpu.async_copy(w_hbm.at[src_v.at[j]], rows_v.at[b], sem)

        def gwait(j, b, sem):
            pltpu.make_async_copy(w_hbm.at[src_v.at[j]], rows_v.at[b], sem).wait()

        def scat(j, b):
            pltpu.sync_copy(rows_v.at[b], acc.at[dst_v.at[j]], add=True)

        npair = nchunk_w // 2
        tail = nchunk_w - 2 * npair
        gather(0, 0, sem_a)

        def body(jj, _):
            j0 = jj * 2
            gather(j0 + 1, 1, sem_b)
            gwait(j0, 0, sem_a)
            scat(j0, 0)

            @pl.when(j0 + 2 < nchunk_w)
            def _():
                gather(j0 + 2, 0, sem_a)

            gwait(j0 + 1, 1, sem_b)
            scat(j0 + 1, 1)
            return 0

        lax.fori_loop(0, npair, body, 0)
        if tail:
            gwait(nchunk_w - 1, 0, sem_a)
            scat(nchunk_w - 1, 0)
        plsc.subcore_barrier()

        for kk in range(nzcopy):
            r = row0 + kk * zrows
            pltpu.sync_copy(acc.at[pl.ds(r, zrows)], out_hbm.at[c, pl.ds(r, zrows)])

    return k(weighted, src2d, dst2d)


def kernel(src_feats, cj, ci, edge_index):
    n, d = src_feats.shape
    e = edge_index.shape[1]
    epw = e // NW
    assert epw * NW == e
    chunk = _pick_chunk(epw)
    nchunk_w = epw // chunk
    zrows = n // NS
    for z in (25, 16, 8, 5, 4, 2, 1):
        if (n // NS) % z == 0:
            zrows = z
            break

    br = 1000 if n % 1000 == 0 else 8

    weighted = pl.pallas_call(
        _mul_rows_kernel,
        out_shape=jax.ShapeDtypeStruct((n, d), jnp.float32),
        grid=(n // br,),
        in_specs=[
            pl.BlockSpec((br, d), lambda i: (i, 0)),
            pl.BlockSpec((br, 1), lambda i: (i, 0)),
        ],
        out_specs=pl.BlockSpec((br, d), lambda i: (i, 0)),
    )(src_feats, cj)

    src2d = edge_index[0].reshape(e // chunk, chunk)
    dst2d = edge_index[1].reshape(e // chunk, chunk)
    partial = _sc_aggregate(weighted, src2d, dst2d, n, d, chunk, nchunk_w, zrows)

    out = pl.pallas_call(
        _combine_kernel,
        out_shape=jax.ShapeDtypeStruct((n, d), jnp.float32),
        grid=(n // br,),
        in_specs=[
            pl.BlockSpec((NC, br, d), lambda i: (0, i, 0)),
            pl.BlockSpec((br, 1), lambda i: (i, 0)),
        ],
        out_specs=pl.BlockSpec((br, d), lambda i: (i, 0)),
    )(partial, ci)
    return out
